# baseline (speedup 1.0000x reference)
import jax
import jax.numpy as jnp
from jax import lax
from jax.experimental import pallas as pl
from jax.experimental.pallas import tpu as pltpu
from jax.experimental.pallas import tpu_sc as plsc

B, F, D = 16384, 100, 64
NC, NS, L = 2, 16, 16
NW = NC * NS
BPW = B // NW
CB = 1
NCH = BPW // CB
XSL = 32
ROWS = XSL * F


def _sc_body(x_hbm, emb_hbm, w_hbm, bias_hbm, out_hbm,
             basebuf, wvbuf, biasbuf, xbuf,
             obuf0, obuf1, sxs, so0, so1):
    wid = lax.axis_index("s") * NC + lax.axis_index("c")
    x0 = wid * (BPW * F)
    b0 = wid * BPW

    pltpu.sync_copy(emb_hbm, basebuf)
    pltpu.sync_copy(w_hbm, wvbuf)
    pltpu.sync_copy(bias_hbm, biasbuf)

    bias_regs = [biasbuf[pl.ds(dc * L, L)] for dc in range(4)]
    wv_regs = [wvbuf[pl.ds(dc * L, L)] for dc in range(4)]

    @plsc.parallel_loop(0, F)
    def _fold(f):
        for dc in range(4):
            o = f * D + dc * L
            basebuf[pl.ds(o, L)] = basebuf[pl.ds(o, L)] + bias_regs[dc]

    obufs = (obuf0, obuf1)
    sos = (so0, so1)

    @pl.loop(0, BPW // XSL)
    def _slab(sl):
        pltpu.async_copy(
            x_hbm.at[pl.ds(x0 + sl * ROWS, ROWS)], xbuf, sxs)
        pltpu.make_async_copy(
            x_hbm.at[pl.ds(x0, ROWS)], xbuf, sxs).wait()

        @pl.loop(0, XSL, step=2)
        def _pair(j):
            for k in range(2):
                bi = j + k
                ob, so = obufs[k], sos[k]
                n = sl * XSL + bi

                @pl.when(n >= 2)
                def _():
                    pltpu.make_async_copy(
                        ob, out_hbm.at[pl.ds(b0, 1)], so).wait()

                row0 = bi * F
                for f in range(F):
                    xs = plsc.load_gather(
                        xbuf, [jnp.full((L,), row0 + f, jnp.int32)])
                    for dc in range(4):
                        ob[0, f, pl.ds(dc * L, L)] = (
                            xs * wv_regs[dc]
                            + basebuf[pl.ds(f * D + dc * L, L)])

                pltpu.async_copy(
                    ob, out_hbm.at[pl.ds(b0 + n, 1)], so)

    pltpu.make_async_copy(obuf0, out_hbm.at[pl.ds(b0, 1)], so0).wait()
    pltpu.make_async_copy(obuf1, out_hbm.at[pl.ds(b0, 1)], so1).wait()


@jax.jit
def kernel(x, emb_table, W, b):
    mesh = plsc.VectorSubcoreMesh(
        core_axis_name="c", subcore_axis_name="s",
        num_cores=NC, num_subcores=NS)
    return pl.kernel(
        _sc_body,
        out_type=jax.ShapeDtypeStruct((B, F, D), jnp.float32),
        mesh=mesh,
        scratch_types=[
            pltpu.VMEM((F * D,), jnp.float32),
            pltpu.VMEM((D,), jnp.float32),
            pltpu.VMEM((D,), jnp.float32),
            pltpu.VMEM((ROWS,), jnp.float32),
            pltpu.VMEM((1, F, D), jnp.float32),
            pltpu.VMEM((1, F, D), jnp.float32),
            pltpu.SemaphoreType.DMA,
            pltpu.SemaphoreType.DMA,
            pltpu.SemaphoreType.DMA,
        ],
        compiler_params=pltpu.CompilerParams(needs_layout_passes=False),
    )(x.reshape(-1), emb_table.reshape(-1), W.reshape(-1), b)


# speedup vs baseline: 1.3614x; 1.3614x over previous
import jax
import jax.numpy as jnp
from jax import lax
from jax.experimental import pallas as pl
from jax.experimental import layout as jex_layout
from jax.experimental.pallas import tpu as pltpu
from jax.experimental.pallas import tpu_sc as plsc

B, F, D = 16384, 100, 64
NC, NS, L = 2, 16, 16
NW = NC * NS
BPW = B // NW
CB = 8
NCH = BPW // CB
ROWS = CB * F


def _sc_body(x_hbm, emb_hbm, w_hbm, bias_hbm, out_hbm,
             basebuf, wvbuf, biasbuf,
             xbuf0, xbuf1, obuf0, obuf1,
             sx0, sx1, so0, so1):
    wid = lax.axis_index("s") * NC + lax.axis_index("c")
    x0 = wid * (BPW * F)
    b0 = wid * BPW

    pltpu.sync_copy(emb_hbm, basebuf)
    pltpu.sync_copy(w_hbm, wvbuf)
    pltpu.sync_copy(bias_hbm, biasbuf)

    bias_regs = [biasbuf[pl.ds(dc * L, L)] for dc in range(4)]
    wv_regs = [wvbuf[pl.ds(dc * L, L)] for dc in range(4)]

    @plsc.parallel_loop(0, F)
    def _fold(f):
        for dc in range(4):
            o = f * D + dc * L
            basebuf[pl.ds(o, L)] = basebuf[pl.ds(o, L)] + bias_regs[dc]

    xbufs = (xbuf0, xbuf1)
    obufs = (obuf0, obuf1)
    sxs = (sx0, sx1)
    sos = (so0, so1)

    pltpu.async_copy(x_hbm.at[pl.ds(x0, ROWS)], xbuf0, sx0)
    pltpu.async_copy(x_hbm.at[pl.ds(x0 + ROWS, ROWS)], xbuf1, sx1)

    @pl.loop(0, NCH, step=2)
    def _chunks(c):
        for k in range(2):
            cc = c + k
            xb, ob, sx, so = xbufs[k], obufs[k], sxs[k], sos[k]
            pltpu.make_async_copy(
                x_hbm.at[pl.ds(x0 + cc * ROWS, ROWS)], xb, sx).wait()

            @pl.when(cc >= 2)
            def _():
                pltpu.make_async_copy(
                    ob, out_hbm.at[pl.ds(b0, CB)], so).wait()

            @pl.loop(0, CB)
            def _batch(bi):
                row0 = bi * F

                @plsc.parallel_loop(0, F, unroll=2)
                def _row(f):
                    xs = plsc.load_gather(
                        xb, [jnp.full((L,), row0 + f, jnp.int32)])
                    for dc in range(4):
                        ob[bi, f, pl.ds(dc * L, L)] = (
                            xs * wv_regs[dc]
                            + basebuf[pl.ds(f * D + dc * L, L)])

            pltpu.async_copy(
                ob, out_hbm.at[pl.ds(b0 + cc * CB, CB)], so)

            @pl.when(cc + 2 < NCH)
            def _():
                pltpu.async_copy(
                    x_hbm.at[pl.ds(x0 + (cc + 2) * ROWS, ROWS)], xb, sx)

    pltpu.make_async_copy(obuf0, out_hbm.at[pl.ds(b0, CB)], so0).wait()
    pltpu.make_async_copy(obuf1, out_hbm.at[pl.ds(b0, CB)], so1).wait()


def _kernel_impl(x, emb_table, W, b):
    mesh = plsc.VectorSubcoreMesh(
        core_axis_name="c", subcore_axis_name="s",
        num_cores=NC, num_subcores=NS)
    return pl.kernel(
        _sc_body,
        out_type=jax.ShapeDtypeStruct((B, F, D), jnp.float32),
        mesh=mesh,
        scratch_types=[
            pltpu.VMEM((F * D,), jnp.float32),
            pltpu.VMEM((D,), jnp.float32),
            pltpu.VMEM((D,), jnp.float32),
            pltpu.VMEM((ROWS,), jnp.float32),
            pltpu.VMEM((ROWS,), jnp.float32),
            pltpu.VMEM((CB, F, D), jnp.float32),
            pltpu.VMEM((CB, F, D), jnp.float32),
            pltpu.SemaphoreType.DMA,
            pltpu.SemaphoreType.DMA,
            pltpu.SemaphoreType.DMA,
            pltpu.SemaphoreType.DMA,
        ],
        compiler_params=pltpu.CompilerParams(
            needs_layout_passes=False,
            use_tc_tiling_on_sc=False,
        ),
    )(x.reshape(-1), emb_table.reshape(-1), W.reshape(-1), b)


_JITTED = None


def kernel(x, emb_table, W, b):
    global _JITTED
    if _JITTED is None:
        fmt = jex_layout.Format(
            jex_layout.Layout(major_to_minor=(0, 1, 2), tiling=((1, 1),)),
            jax.sharding.SingleDeviceSharding(jax.devices()[0]))
        _JITTED = jax.jit(_kernel_impl, out_shardings=fmt)
    return _JITTED(x, emb_table, W, b)
